# baseline (speedup 1.0000x reference)
import jax
import jax.numpy as jnp
from jax.experimental import pallas as pl
from jax.experimental.pallas import tpu as pltpu

_MU = 0.02
_BINS = 10
_ALPHA = 0.75
_N = 8388608

_COLS = 1024
_ROWS = _N // _COLS
_BLK = 512
_GRID = _ROWS // _BLK
_NACC = 21


def _body(p_ref, t_ref, out_ref, acc_ref):
    step = pl.program_id(0)

    @pl.when(step == 0)
    def _init():
        acc_ref[...] = jnp.zeros_like(acc_ref)

    p = p_ref[...]
    t = t_ref[...]
    d = jnp.abs(p - t)
    loss = jnp.where(d < _MU, (0.5 / _MU) * d * d, d - 0.5 * _MU)
    m = jnp.abs(jnp.tanh(p) - jnp.tanh(t)) * float(_BINS)

    acc_ref[0, :] += jnp.sum(loss, axis=0)
    for k in range(1, _BINS):
        mask = m >= float(k)
        acc_ref[k, :] += jnp.sum(jnp.where(mask, loss, 0.0), axis=0)
        acc_ref[_BINS + k, :] += jnp.sum(jnp.where(mask, 1.0, 0.0), axis=0)
    acc_ref[2 * _BINS, :] += jnp.sum(
        jnp.where(m >= float(_BINS), 1.0, 0.0), axis=0)

    @pl.when(step == _GRID - 1)
    def _finish():
        ls_ge = [jnp.sum(acc_ref[k, :]) for k in range(_BINS)]
        cnt_ge = [jnp.float32(_N)] + [
            jnp.sum(acc_ref[_BINS + k, :]) for k in range(1, _BINS + 1)]
        lanes = jax.lax.broadcasted_iota(jnp.int32, (8, 128), 1) + \
            128 * jax.lax.broadcasted_iota(jnp.int32, (8, 128), 0)
        tot_v = jnp.ones((8, 128), jnp.float32)
        ls_v = jnp.zeros((8, 128), jnp.float32)
        for b in range(_BINS):
            cnt_b = cnt_ge[b] - cnt_ge[b + 1]
            ls_b = ls_ge[b] - (ls_ge[b + 1] if b + 1 < _BINS else 0.0)
            tot_v = jnp.where(lanes == b, jnp.maximum(cnt_b, 1.0), tot_v)
            ls_v = jnp.where(lanes == b, ls_b, ls_v)
        w_v = jnp.exp(-_ALPHA * jnp.log(tot_v))
        out_ref[0, 0] = jnp.sum(ls_v * w_v) * (1.0 / _N)


def kernel(pred, target):
    p2 = pred.reshape(_ROWS, _COLS)
    t2 = target.reshape(_ROWS, _COLS)
    out = pl.pallas_call(
        _body,
        grid=(_GRID,),
        in_specs=[
            pl.BlockSpec((_BLK, _COLS), lambda i: (i, 0)),
            pl.BlockSpec((_BLK, _COLS), lambda i: (i, 0)),
        ],
        out_specs=pl.BlockSpec(memory_space=pltpu.SMEM),
        out_shape=jax.ShapeDtypeStruct((1, 1), jnp.float32),
        scratch_shapes=[pltpu.VMEM((_NACC + 3, _COLS), jnp.float32)],
        compiler_params=pltpu.CompilerParams(
            dimension_semantics=("arbitrary",)),
    )(p2, t2)
    return out[0, 0]
